# Initial kernel scaffold; baseline (speedup 1.0000x reference)
import jax
import jax.numpy as jnp
from jax.experimental import pallas as pl


def kernel(feat_p3, feat_p4, feat_p5, w_stem, b_stem, w_obj, b_obj, w_box, b_box):
    raise NotImplementedError("write your pallas kernel here")


# speedup vs baseline: 7.9448x; 7.9448x over previous
import functools
import math

import jax
import jax.numpy as jnp
from jax.experimental import pallas as pl
from jax.experimental.pallas import tpu as pltpu

_STRIDES = (8, 16, 32)
_HW = ((64, 64), (32, 32), (16, 16))
_A = 3
_NMS_T = 0.7
_K_PRE = 400
_K_POST = 100
_SCALE_CLAMP = math.log(224.0 / 8.0)
_C = 128
_HI = jax.lax.Precision.HIGHEST


def _bdot(a, b):
    return jax.lax.dot(a.astype(jnp.bfloat16), b.astype(jnp.bfloat16),
                       preferred_element_type=jnp.float32)


def _head_body(H, W, x_ref, w9_ref, bs_ref, wh_ref, bh_ref, out_ref):
    acc = jnp.zeros((H * W, _C), jnp.float32)
    for k in range(9):
        ky, kx = divmod(k, 3)
        xk = x_ref[0, ky:ky + H, kx:kx + W, :].reshape(H * W, _C)
        acc += _bdot(xk, w9_ref[k])
    t = jnp.maximum(acc + bs_ref[0], 0.0)
    out_ref[0] = _bdot(t, wh_ref[...]) + bh_ref[0]


def _heads(x, w9, bs, wh, bh, H, W):
    B = x.shape[0]
    xp = jnp.pad(x.transpose(0, 2, 3, 1), ((0, 0), (1, 1), (1, 1), (0, 0)))
    return pl.pallas_call(
        functools.partial(_head_body, H, W),
        grid=(B,),
        in_specs=[
            pl.BlockSpec((1, H + 2, W + 2, _C), lambda b: (b, 0, 0, 0)),
            pl.BlockSpec((9, _C, _C), lambda b: (0, 0, 0)),
            pl.BlockSpec((1, _C), lambda b: (0, 0)),
            pl.BlockSpec((_C, 16), lambda b: (0, 0)),
            pl.BlockSpec((1, 16), lambda b: (0, 0)),
        ],
        out_specs=pl.BlockSpec((1, H * W, 16), lambda b: (b, 0, 0)),
        out_shape=jax.ShapeDtypeStruct((B, H * W, 16), jnp.float32),
    )(xp, w9, bs, wh, bh)


def _col(v):
    return jnp.transpose(v, (1, 0))


def _decode_level(l, sl, nl, d):
    stride = _STRIDES[l]
    Wl = _HW[l][1]
    hw = nl // _A
    a = nl - hw * _A
    i = hw // Wl
    j = hw - i * Wl
    px = stride * (i.astype(jnp.float32) + 0.5)
    py = stride * (j.astype(jnp.float32) + 0.5)
    area = float((8 * stride) ** 2)
    dims = []
    for ar in (0.5, 1.0, 2.0):
        nw = math.sqrt(area / ar)
        dims.append((nw, area / nw))
    ph = jnp.where(a == 0, dims[0][0], jnp.where(a == 1, dims[1][0], dims[2][0]))
    pw = jnp.where(a == 0, dims[0][1], jnp.where(a == 1, dims[1][1], dims[2][1]))
    dx = d[0:1]
    dy = d[1:2]
    dw = jnp.minimum(d[2:3], _SCALE_CLAMP)
    dh = jnp.minimum(d[3:4], _SCALE_CLAMP)
    bx = px + ph * dx
    by = py + pw * dy
    bh2 = ph * jnp.exp(dw) * 0.5
    bw2 = pw * jnp.exp(dh) * 0.5
    x0 = bx - bh2
    y0 = by - bw2
    x1 = bx + bh2
    y1 = by + bw2
    sent = dx == 1e-08
    x0 = jnp.where(sent, 1e-08, x0)
    y0 = jnp.where(sent, 1e-08, y0)
    x1 = jnp.where(sent, 1e-08, x1)
    y1 = jnp.where(sent, 1e-08, y1)
    return x0, y0, x1, y1


def _nms_keep(x0, y0, x1, y1):
    K = x0.shape[1]
    xA = jnp.maximum(_col(x0), x0)
    yA = jnp.maximum(_col(y0), y0)
    xB = jnp.minimum(_col(x1), x1)
    yB = jnp.minimum(_col(y1), y1)
    inter = jnp.maximum(xB - xA, 0.0) * jnp.maximum(yB - yA, 0.0)
    ar = (x1 - x0) * (y1 - y0)
    iou = inter / (_col(ar) + ar - inter)
    rI = jax.lax.broadcasted_iota(jnp.int32, (K, K), 0)
    cI = jax.lax.broadcasted_iota(jnp.int32, (K, K), 1)
    M = ((iou > _NMS_T) & (rI < cI)).astype(jnp.float32)

    def cond(c):
        return c[1]

    def body(c):
        keep = c[0]
        supp = jax.lax.dot(keep, M) > 0.5
        new = jnp.where(supp, 0.0, 1.0)
        return new, jnp.any(new != keep)

    keep, _ = jax.lax.while_loop(
        cond, body, (jnp.ones((1, K), jnp.float32), jnp.bool_(True)))
    return keep


def _nms_body(ts_ref, ti_ref, td_ref, out_ref, sel_ref):
    level_boxes = []
    level_scores = []
    for l in range(3):
        sl = ts_ref[0, l:l + 1, :]
        nl = ti_ref[0, l:l + 1, :]
        d = td_ref[0, l]
        x0, y0, x1, y1 = _decode_level(l, sl, nl, d)
        keep = _nms_keep(x0, y0, x1, y1)
        sc = jnp.where(keep > 0.5, jax.nn.sigmoid(sl), -1.0)
        level_scores.append(sc)
        level_boxes.append(jnp.concatenate([x0, y0, x1, y1], axis=0))
    allS = jnp.concatenate(level_scores, axis=1)
    allB = jnp.concatenate(level_boxes, axis=1)
    data = jnp.concatenate([allB, allS], axis=0)
    n_all = allS.shape[1]
    lane = jax.lax.broadcasted_iota(jnp.int32, (1, n_all), 1)

    def fbody(j, cur):
        m = jnp.max(cur)
        pi = jnp.min(jnp.where(cur == m, lane, jnp.int32(1 << 20)))
        onehot = lane == pi
        sel_ref[pl.ds(j, 1), :] = onehot.astype(jnp.float32)
        return jnp.where(onehot, -jnp.inf, cur)

    jax.lax.fori_loop(0, _K_POST, fbody, allS)
    out_ref[0] = jax.lax.dot_general(
        data, sel_ref[...], (((1,), (1,)), ((), ())), precision=_HI)


def kernel(feat_p3, feat_p4, feat_p5, w_stem, b_stem, w_obj, b_obj, w_box,
           b_box):
    B = feat_p3.shape[0]
    w9 = w_stem.transpose(2, 3, 1, 0).reshape(9, _C, _C)
    wh = jnp.concatenate(
        [w_obj[:, :, 0, 0].T, w_box[:, :, 0, 0].T,
         jnp.zeros((_C, 1), jnp.float32)], axis=1)
    bh = jnp.concatenate([b_obj, b_box, jnp.zeros((1,), jnp.float32)])[None]
    bs = b_stem[None]
    ts_l, ti_l, td_l = [], [], []
    for l, x in enumerate((feat_p3, feat_p4, feat_p5)):
        H, W = _HW[l]
        out = _heads(x, w9, bs, wh, bh, H, W)
        obj = out[:, :, :3].reshape(B, H * W * _A)
        dl = out[:, :, 3:15].reshape(B, H * W * _A, 4)
        s, i = jax.lax.top_k(obj, _K_PRE)
        d = jnp.take_along_axis(dl, i[:, :, None], axis=1)
        ts_l.append(s)
        ti_l.append(i)
        td_l.append(d.transpose(0, 2, 1))
    ts = jnp.stack(ts_l, 1)
    ti = jnp.stack(ti_l, 1)
    td = jnp.stack(td_l, 1)
    out5 = pl.pallas_call(
        _nms_body,
        grid=(B,),
        in_specs=[
            pl.BlockSpec((1, 3, _K_PRE), lambda b: (b, 0, 0)),
            pl.BlockSpec((1, 3, _K_PRE), lambda b: (b, 0, 0)),
            pl.BlockSpec((1, 3, 4, _K_PRE), lambda b: (b, 0, 0, 0)),
        ],
        out_specs=pl.BlockSpec((1, 5, _K_POST), lambda b: (b, 0, 0)),
        out_shape=jax.ShapeDtypeStruct((B, 5, _K_POST), jnp.float32),
        scratch_shapes=[pltpu.VMEM((_K_POST, 3 * _K_PRE), jnp.float32)],
    )(ts, ti, td)
    return out5.transpose(0, 2, 1)
